# baseline (speedup 1.0000x reference)
import functools

import numpy as np
import jax
import jax.numpy as jnp
from jax import lax
from jax.experimental import pallas as pl
from jax.experimental.pallas import tpu as pltpu
from jax.experimental.pallas import tpu_sc as plsc

_VOCAB = 100000
_D = 128
_B = 1024
_S = 200
_N = _B * _S
_NC = 2
_NS = 16
_NW = _NC * _NS
_PER_W = _N // _NW
_CHUNK = 128
_NCHUNK = _PER_W // _CHUNK
_RING = 5


def _positional_encoding_np(seq_len, d_model):
    positions = np.arange(seq_len)
    dimensions = np.arange(d_model)
    denominator = np.power(10000.0, 2 * dimensions / d_model)
    input_angles = positions.reshape(-1, 1) / denominator.reshape(1, -1)
    pe = np.zeros(shape=input_angles.shape)
    pe[:, 0::2] = np.sin(input_angles[:, 0::2])
    pe[:, 1::2] = np.cos(input_angles[:, 1::2])
    return pe.astype(np.float32)


_PE_FLAT_NP = _positional_encoding_np(_S, _D).reshape(-1)


_mesh = plsc.VectorSubcoreMesh(core_axis_name="c", subcore_axis_name="s")


@functools.partial(
    pl.kernel,
    mesh=_mesh,
    out_type=jax.ShapeDtypeStruct((_N, _D), jnp.float32),
    scratch_types=(
        [pltpu.VMEM((_S * _D,), jnp.float32)]
        + [pltpu.VMEM((_PER_W,), jnp.int32)]
        + [pltpu.VMEM((_CHUNK, _D), jnp.float32)] * _RING
        + [pltpu.SemaphoreType.DMA] * (2 * _RING)
    ),
)
def _emb_lookup(x_hbm, pe_hbm, table_hbm, out_hbm, pe_v, idx_v, *refs):
    rowsb = refs[0:_RING]
    gsb = refs[_RING:2 * _RING]
    osb = refs[2 * _RING:3 * _RING]

    wid = lax.axis_index("s") * _NC + lax.axis_index("c")
    base = wid * _PER_W
    pltpu.sync_copy(pe_hbm, pe_v)

    pltpu.sync_copy(x_hbm.at[pl.ds(base, _PER_W)], idx_v)
    for k in range(3):
        pltpu.async_copy(
            table_hbm.at[idx_v.at[pl.ds(k * _CHUNK, _CHUNK)]],
            rowsb[k], gsb[k])

    def do_chunk(c, v):
        off = base + c * _CHUNK
        iv = c * _CHUNK
        g3 = (v + 3) % _RING

        @pl.when(c + 3 < _NCHUNK)
        def _():
            @pl.when(c >= 2)
            def _():
                pltpu.make_async_copy(
                    rowsb[g3], out_hbm.at[pl.ds(off - 2 * _CHUNK, _CHUNK)],
                    osb[g3]).wait()

            pltpu.async_copy(
                table_hbm.at[idx_v.at[pl.ds(iv + 3 * _CHUNK, _CHUNK)]],
                rowsb[g3], gsb[g3])

        pltpu.make_async_copy(
            table_hbm.at[idx_v.at[pl.ds(iv, _CHUNK)]], rowsb[v],
            gsb[v]).wait()

        zmin = idx_v[pl.ds(iv, 16)]
        for rg in range(1, _CHUNK // 16):
            zmin = jnp.minimum(zmin, idx_v[pl.ds(iv + rg * 16, 16)])
        any_pad = zmin[0] == 0
        for lane in range(1, 16):
            any_pad = jnp.logical_or(any_pad, zmin[lane] == 0)

        @pl.when(any_pad)
        def _():
            def fix_body(rg, fcarry):
                iv16 = idx_v[pl.ds(iv + rg * 16, 16)]
                for lane in range(16):
                    @pl.when(iv16[lane] == 0)
                    def _():
                        r = rg * 16 + lane

                        def zg(g, zc):
                            rowsb[v][r, pl.ds(g * 16, 16)] = jnp.zeros(
                                (16,), jnp.float32)
                            return zc

                        lax.fori_loop(0, _D // 16, zg, 0)
                return fcarry

            lax.fori_loop(0, _CHUNK // 16, fix_body, 0)

        s0 = lax.rem(off, _S)

        @plsc.parallel_loop(0, _CHUNK, unroll=8)
        def add_body(r):
            t = s0 + r
            s = jnp.where(t >= _S, t - _S, t)
            for g in range(_D // 16):
                v16 = rowsb[v][r, pl.ds(g * 16, 16)]
                p16 = pe_v[pl.ds(s * _D + g * 16, 16)]
                rowsb[v][r, pl.ds(g * 16, 16)] = v16 + p16

        pltpu.async_copy(rowsb[v], out_hbm.at[pl.ds(off, _CHUNK)], osb[v])

    def ring_body(p, carry):
        for j in range(_RING):
            do_chunk(_RING * p + j, j)
        return carry

    lax.fori_loop(0, _NCHUNK // _RING, ring_body, 0)

    for k in range(_NCHUNK - _RING, _NCHUNK):
        pltpu.make_async_copy(
            rowsb[k % _RING],
            out_hbm.at[pl.ds(base + k * _CHUNK, _CHUNK)],
            osb[k % _RING]).wait()


def kernel(x, table):
    x_flat = x.reshape(-1).astype(jnp.int32)
    out = _emb_lookup(x_flat, jnp.asarray(_PE_FLAT_NP), table)
    return out.reshape(_B, _S, _D)


# speedup vs baseline: 1.0428x; 1.0428x over previous
import functools

import numpy as np
import jax
import jax.numpy as jnp
from jax import lax
from jax.experimental import pallas as pl
from jax.experimental.pallas import tpu as pltpu
from jax.experimental.pallas import tpu_sc as plsc

_VOCAB = 100000
_D = 128
_B = 1024
_S = 200
_N = _B * _S
_NC = 2
_NS = 16
_NW = _NC * _NS
_PER_W = _N // _NW
_CHUNK = 128
_NCHUNK = _PER_W // _CHUNK
_RING = 4


def _positional_encoding_np(seq_len, d_model):
    positions = np.arange(seq_len)
    dimensions = np.arange(d_model)
    denominator = np.power(10000.0, 2 * dimensions / d_model)
    input_angles = positions.reshape(-1, 1) / denominator.reshape(1, -1)
    pe = np.zeros(shape=input_angles.shape)
    pe[:, 0::2] = np.sin(input_angles[:, 0::2])
    pe[:, 1::2] = np.cos(input_angles[:, 1::2])
    return pe.astype(np.float32)


_PE_FLAT_NP = _positional_encoding_np(_S, _D).reshape(-1)


_mesh = plsc.VectorSubcoreMesh(core_axis_name="c", subcore_axis_name="s")


@functools.partial(
    pl.kernel,
    mesh=_mesh,
    out_type=jax.ShapeDtypeStruct((_N, _D), jnp.float32),
    scratch_types=(
        [pltpu.VMEM((_S * _D,), jnp.float32)]
        + [pltpu.VMEM((_CHUNK,), jnp.int32)] * _RING
        + [pltpu.VMEM((_CHUNK, _D), jnp.float32)] * _RING
        + [pltpu.SemaphoreType.DMA] * (3 * _RING + 1)
    ),
)
def _emb_lookup(x_hbm, pe_hbm, table_hbm, out_hbm, pe_v, *refs):
    idxb = refs[0:_RING]
    rowsb = refs[_RING:2 * _RING]
    isb = refs[2 * _RING:3 * _RING]
    gsb = refs[3 * _RING:4 * _RING]
    osb = refs[4 * _RING:5 * _RING]
    psem = refs[5 * _RING]

    wid = lax.axis_index("s") * _NC + lax.axis_index("c")
    base = wid * _PER_W

    pltpu.async_copy(pe_hbm, pe_v, psem)
    for k in range(3):
        pltpu.async_copy(
            x_hbm.at[pl.ds(base + k * _CHUNK, _CHUNK)], idxb[k], isb[k])
    for k in range(2):
        pltpu.make_async_copy(
            x_hbm.at[pl.ds(base + k * _CHUNK, _CHUNK)], idxb[k],
            isb[k]).wait()
        pltpu.async_copy(table_hbm.at[idxb[k]], rowsb[k], gsb[k])
    pltpu.make_async_copy(pe_hbm, pe_v, psem).wait()

    def do_chunk(c, v):
        off = base + c * _CHUNK
        g2 = (v + 2) % _RING
        g3 = (v + 3) % _RING

        @pl.when(c + 2 < _NCHUNK)
        def _():
            pltpu.make_async_copy(
                x_hbm.at[pl.ds(off + 2 * _CHUNK, _CHUNK)], idxb[g2],
                isb[g2]).wait()

            @pl.when(c >= 2)
            def _():
                pltpu.make_async_copy(
                    rowsb[g2], out_hbm.at[pl.ds(off - 2 * _CHUNK, _CHUNK)],
                    osb[g2]).wait()

            pltpu.async_copy(table_hbm.at[idxb[g2]], rowsb[g2], gsb[g2])

        @pl.when(c + 3 < _NCHUNK)
        def _():
            pltpu.async_copy(
                x_hbm.at[pl.ds(off + 3 * _CHUNK, _CHUNK)], idxb[g3], isb[g3])

        pltpu.make_async_copy(table_hbm.at[idxb[v]], rowsb[v], gsb[v]).wait()

        zmin = idxb[v][pl.ds(0, 16)]
        for rg in range(1, _CHUNK // 16):
            zmin = jnp.minimum(zmin, idxb[v][pl.ds(rg * 16, 16)])
        any_pad = zmin[0] == 0
        for lane in range(1, 16):
            any_pad = jnp.logical_or(any_pad, zmin[lane] == 0)

        @pl.when(any_pad)
        def _():
            def fix_body(rg, fcarry):
                iv16 = idxb[v][pl.ds(rg * 16, 16)]
                for lane in range(16):
                    @pl.when(iv16[lane] == 0)
                    def _():
                        r = rg * 16 + lane

                        def zg(g, zc):
                            rowsb[v][r, pl.ds(g * 16, 16)] = jnp.zeros(
                                (16,), jnp.float32)
                            return zc

                        lax.fori_loop(0, _D // 16, zg, 0)
                return fcarry

            lax.fori_loop(0, _CHUNK // 16, fix_body, 0)

        s0 = lax.rem(off, _S)

        @plsc.parallel_loop(0, _CHUNK, unroll=8)
        def add_body(r):
            t = s0 + r
            s = jnp.where(t >= _S, t - _S, t)
            for g in range(_D // 16):
                v16 = rowsb[v][r, pl.ds(g * 16, 16)]
                p16 = pe_v[pl.ds(s * _D + g * 16, 16)]
                rowsb[v][r, pl.ds(g * 16, 16)] = v16 + p16

        pltpu.async_copy(rowsb[v], out_hbm.at[pl.ds(off, _CHUNK)], osb[v])

    def quad_body(p, carry):
        for j in range(_RING):
            c = _RING * p + j

            @pl.when(c < _NCHUNK)
            def _():
                do_chunk(c, j)
        return carry

    lax.fori_loop(0, (_NCHUNK + _RING - 1) // _RING, quad_body, 0)

    for k in range(_NCHUNK - 4, _NCHUNK):
        pltpu.make_async_copy(
            rowsb[k % _RING],
            out_hbm.at[pl.ds(base + k * _CHUNK, _CHUNK)],
            osb[k % _RING]).wait()


def kernel(x, table):
    x_flat = x.reshape(-1).astype(jnp.int32)
    out = _emb_lookup(x_flat, jnp.asarray(_PE_FLAT_NP), table)
    return out.reshape(_B, _S, _D)
